# Initial kernel scaffold; baseline (speedup 1.0000x reference)
import jax
import jax.numpy as jnp
from jax.experimental import pallas as pl


def kernel(x8, x6, nbr8, nbr6, down_idx, params):
    raise NotImplementedError("write your pallas kernel here")


# speedup vs baseline: 4.5617x; 4.5617x over previous
import functools

import jax
import jax.numpy as jnp
from jax import lax
from jax.experimental import pallas as pl
from jax.experimental.pallas import tpu as pltpu
from jax.experimental.pallas import tpu_sc as plsc

_NUM_CORES = 2
_NUM_SUBCORES = 16
_NW = _NUM_CORES * _NUM_SUBCORES

_K = 27


def _pick_chunk(b_per_w: int, row_bytes: int, budget: int = 352 * 1024) -> int:
    cap = max(8, budget // row_bytes)
    best = 8
    for c in range(8, b_per_w + 1, 8):
        if c > cap:
            break
        if b_per_w % c == 0:
            best = c
    return best


def _sc_gather(table, idx_flat):
    V, D = table.shape
    B = idx_flat.shape[0]
    assert B % (8 * _NW) == 0, (B,)
    b_per_w = B // _NW
    chunk = _pick_chunk(b_per_w, D * 4)
    nchunks = b_per_w // chunk
    mesh = plsc.VectorSubcoreMesh(
        core_axis_name="c", subcore_axis_name="s",
        num_cores=_NUM_CORES, num_subcores=_NUM_SUBCORES)

    @functools.partial(
        pl.kernel,
        out_type=jax.ShapeDtypeStruct((B, D), jnp.float32),
        mesh=mesh,
        scratch_types=[
            pltpu.VMEM((chunk,), jnp.int32),
            pltpu.VMEM((chunk, D), jnp.float32),
            pltpu.SemaphoreType.DMA,
        ],
        compiler_params=pltpu.CompilerParams(use_tc_tiling_on_sc=False),
    )
    def gather_kernel(table_hbm, idx_hbm, out_hbm, idx_v, rows_v, sem):
        wid = lax.axis_index("s") * _NUM_CORES + lax.axis_index("c")
        base = pl.multiple_of(wid * b_per_w, 8)

        @pl.loop(0, nchunks)
        def _chunk_loop(c):
            off = pl.multiple_of(base + c * chunk, 8)
            pltpu.sync_copy(idx_hbm.at[pl.ds(off, chunk)], idx_v)
            pltpu.async_copy(table_hbm.at[idx_v], rows_v, sem).wait()
            pltpu.sync_copy(rows_v, out_hbm.at[pl.ds(off, chunk)])

    return gather_kernel(table, idx_flat)


def _conv_tc(g2d, w_flat, bias, gamma, beta, block, leaky, ln):
    npad, kd = g2d.shape
    out_dim = w_flat.shape[1]
    grid = (npad // block,)

    def body(*refs):
        if ln:
            g_ref, w_ref, b_ref, gam_ref, bet_ref, o_ref = refs
        else:
            g_ref, w_ref, b_ref, o_ref = refs
        x = jnp.dot(g_ref[...], w_ref[...], preferred_element_type=jnp.float32)
        x = x + b_ref[...]
        if leaky:
            x = jnp.where(x >= 0, x, 0.2 * x)
        if ln:
            m = jnp.mean(x, axis=-1, keepdims=True)
            v = jnp.mean((x - m) * (x - m), axis=-1, keepdims=True)
            x = (x - m) * lax.rsqrt(v + 1e-5) * gam_ref[...] + bet_ref[...]
        o_ref[...] = x

    in_specs = [
        pl.BlockSpec((block, kd), lambda i: (i, 0)),
        pl.BlockSpec((kd, out_dim), lambda i: (0, 0)),
        pl.BlockSpec((1, out_dim), lambda i: (0, 0)),
    ]
    args = [g2d, w_flat, bias]
    if ln:
        in_specs += [pl.BlockSpec((1, out_dim), lambda i: (0, 0))] * 2
        args += [gamma, beta]
    return pl.pallas_call(
        body,
        grid=grid,
        in_specs=in_specs,
        out_specs=pl.BlockSpec((block, out_dim), lambda i: (i, 0)),
        out_shape=jax.ShapeDtypeStruct((npad, out_dim), jnp.float32),
    )(*args)


def _pad_rows(a, npad):
    n = a.shape[0]
    if n == npad:
        return a
    return jnp.pad(a, ((0, npad - n),) + ((0, 0),) * (a.ndim - 1))


def _layer(x, idx_flat, npad, w, b, g=None, beta=None, block=512,
           leaky=True, ln=False):
    k, din, out_dim = w.shape
    gr = _sc_gather(x, idx_flat)
    g2d = gr.reshape(npad, k * din)
    return _conv_tc(
        g2d, w.reshape(k * din, out_dim), b.reshape(1, out_dim),
        None if g is None else g.reshape(1, out_dim),
        None if beta is None else beta.reshape(1, out_dim),
        block, leaky, ln)


def kernel(x8, x6, nbr8, nbr6, down_idx, params):
    p = params
    n8, n6 = x8.shape[0], x6.shape[0]
    npad8 = ((n8 + 511) // 512) * 512
    npad6 = ((n6 + 511) // 512) * 512

    idx8 = _pad_rows(nbr8, npad8).reshape(-1)
    idx6 = _pad_rows(nbr6, npad6).reshape(-1)
    idxd = _pad_rows(down_idx, npad6).reshape(-1)

    h = _layer(x8, idx8, npad8, p['init0_W'], p['init0_b'])
    h = _layer(h, idx8, npad8, p['proc0_W'], p['proc0_b'],
               p['proc0_g'], p['proc0_beta'], ln=True)
    prev = _layer(h, idx8, npad8, p['proc1_W'], p['proc1_b'],
                  p['proc1_g'], p['proc1_beta'], ln=True)

    out6 = _layer(x6, idx6, npad6, p['init1_W'], p['init1_b'])
    down = _layer(prev, idxd, npad6, p['down0_W'], p['down0_b'])
    h6 = jnp.concatenate([down, out6], axis=1)
    h6 = _layer(h6, idx6, npad6, p['proc2_W'], p['proc2_b'],
                p['proc2_g'], p['proc2_beta'], ln=True)
    h6 = _layer(h6, idx6, npad6, p['proc3_W'], p['proc3_b'],
                p['proc3_g'], p['proc3_beta'], ln=True)
    out = _layer(h6, idx6, npad6, p['head_W'], p['head_b'], leaky=False)
    return out[:n6]
